# initial kernel scaffold (unmeasured)
import jax
import jax.numpy as jnp
from jax import lax
from jax.experimental import pallas as pl
from jax.experimental.pallas import tpu as pltpu

N_DEV = 32
M = 2048
N = 2048
CHUNK = M // N_DEV


def kernel(A, B):
    k = A.shape[1]

    def body(a_ref, b_ref, out_ref, a16_ref, b16_ref, comm_ref,
             send_sems, recv_sems, credit_sem):
        my = lax.axis_index("i")
        left = lax.rem(my - 1 + N_DEV, N_DEV)
        right = lax.rem(my + 1, N_DEV)

        barrier = pltpu.get_barrier_semaphore()
        for nbr in (left, right):
            pl.semaphore_signal(barrier, inc=1, device_id=(nbr,),
                                device_id_type=pl.DeviceIdType.MESH)
        pl.semaphore_wait(barrier, 2)

        a16_ref[:, :] = a_ref[:, :].astype(jnp.bfloat16)
        b16_ref[:, :] = b_ref[:, :].astype(jnp.bfloat16)

        def partial_chunk(c):
            rows = a16_ref[pl.ds(c * CHUNK, CHUNK), :]
            return jnp.dot(rows, b16_ref[:, :],
                           preferred_element_type=jnp.float32)

        comm_ref[1, :, :] = partial_chunk(my)

        LAST = 2 * N_DEV - 2
        for H in range(LAST):
            send_slot = (H + 1) % 2
            recv_slot = H % 2
            if H >= 2:
                pl.semaphore_wait(credit_sem, 1)
            rdma = pltpu.make_async_remote_copy(
                src_ref=comm_ref.at[send_slot],
                dst_ref=comm_ref.at[recv_slot],
                send_sem=send_sems.at[send_slot],
                recv_sem=recv_sems.at[recv_slot],
                device_id=(right,),
                device_id_type=pl.DeviceIdType.MESH,
            )
            rdma.start()
            if H < N_DEV - 1:
                rc = lax.rem(my - H - 1 + N_DEV, N_DEV)
                p = partial_chunk(rc)
                rdma.wait_recv()
                acc = comm_ref[recv_slot, :, :] + p
                if H == N_DEV - 2:
                    acc = jnp.maximum(acc, 0.0)
                    o = lax.rem(my + 1, N_DEV)
                    out_ref[pl.ds(o * CHUNK, CHUNK), :] = acc
                comm_ref[recv_slot, :, :] = acc
            else:
                g = H - (N_DEV - 1)
                rdma.wait_recv()
                c = lax.rem(my - g + N_DEV, N_DEV)
                out_ref[pl.ds(c * CHUNK, CHUNK), :] = comm_ref[recv_slot, :, :]
            rdma.wait_send()
            if 1 <= H <= LAST - 2:
                pl.semaphore_signal(credit_sem, inc=1, device_id=(left,),
                                    device_id_type=pl.DeviceIdType.MESH)

    return pl.pallas_call(
        body,
        out_shape=jax.ShapeDtypeStruct((M, N), jnp.float32),
        in_specs=[
            pl.BlockSpec(memory_space=pltpu.VMEM),
            pl.BlockSpec(memory_space=pltpu.VMEM),
        ],
        out_specs=pl.BlockSpec(memory_space=pltpu.VMEM),
        scratch_shapes=[
            pltpu.VMEM((M, k), jnp.bfloat16),
            pltpu.VMEM((k, N), jnp.bfloat16),
            pltpu.VMEM((2, CHUNK, N), jnp.float32),
            pltpu.SemaphoreType.DMA((2,)),
            pltpu.SemaphoreType.DMA((2,)),
            pltpu.SemaphoreType.REGULAR,
        ],
        compiler_params=pltpu.CompilerParams(collective_id=0),
    )(A, B)


# baseline (device time: 818352 ns/iter reference)
import jax
import jax.numpy as jnp
from jax import lax
from jax.experimental import pallas as pl
from jax.experimental.pallas import tpu as pltpu

N_DEV = 32
M = 2048
N = 2048
CHUNK = M // N_DEV


def kernel(A, B):
    k = A.shape[1]

    def body(a_ref, b_ref, out_ref, a16_ref, b16_ref, comm_ref,
             send_sems, recv_sems, credit_sem):
        my = lax.axis_index("i")
        left = lax.rem(my - 1 + N_DEV, N_DEV)
        right = lax.rem(my + 1, N_DEV)

        barrier = pltpu.get_barrier_semaphore()
        for nbr in (left, right):
            pl.semaphore_signal(barrier, inc=1, device_id=(nbr,),
                                device_id_type=pl.DeviceIdType.MESH)
        pl.semaphore_wait(barrier, 2)

        a16_ref[:, :] = a_ref[:, :].astype(jnp.bfloat16)
        b16_ref[:, :] = b_ref[:, :].astype(jnp.bfloat16)

        def partial_chunk(c):
            rows = a16_ref[pl.ds(c * CHUNK, CHUNK), :]
            return jnp.dot(rows, b16_ref[:, :],
                           preferred_element_type=jnp.float32)

        comm_ref[1, :, :] = partial_chunk(my)

        LAST = 2 * N_DEV - 2
        for H in range(LAST):
            send_slot = (H + 1) % 2
            recv_slot = H % 2
            if H >= 1:
                pl.semaphore_wait(credit_sem, 1)
            rdma = pltpu.make_async_remote_copy(
                src_ref=comm_ref.at[send_slot],
                dst_ref=comm_ref.at[recv_slot],
                send_sem=send_sems.at[send_slot],
                recv_sem=recv_sems.at[recv_slot],
                device_id=(right,),
                device_id_type=pl.DeviceIdType.MESH,
            )
            rdma.start()
            if H < N_DEV - 1:
                rc = lax.rem(my - H - 1 + N_DEV, N_DEV)
                p = partial_chunk(rc)
                rdma.wait_recv()
                acc = comm_ref[recv_slot, :, :] + p
                if H == N_DEV - 2:
                    acc = jnp.maximum(acc, 0.0)
                    o = lax.rem(my + 1, N_DEV)
                    out_ref[pl.ds(o * CHUNK, CHUNK), :] = acc
                comm_ref[recv_slot, :, :] = acc
            else:
                g = H - (N_DEV - 1)
                rdma.wait_recv()
                c = lax.rem(my - g + N_DEV, N_DEV)
                out_ref[pl.ds(c * CHUNK, CHUNK), :] = comm_ref[recv_slot, :, :]
            rdma.wait_send()
            if H <= LAST - 2:
                pl.semaphore_signal(credit_sem, inc=1, device_id=(left,),
                                    device_id_type=pl.DeviceIdType.MESH)

    return pl.pallas_call(
        body,
        out_shape=jax.ShapeDtypeStruct((M, N), jnp.float32),
        in_specs=[
            pl.BlockSpec(memory_space=pltpu.VMEM),
            pl.BlockSpec(memory_space=pltpu.VMEM),
        ],
        out_specs=pl.BlockSpec(memory_space=pltpu.VMEM),
        scratch_shapes=[
            pltpu.VMEM((M, k), jnp.bfloat16),
            pltpu.VMEM((k, N), jnp.bfloat16),
            pltpu.VMEM((2, CHUNK, N), jnp.float32),
            pltpu.SemaphoreType.DMA((2,)),
            pltpu.SemaphoreType.DMA((2,)),
            pltpu.SemaphoreType.REGULAR,
        ],
        compiler_params=pltpu.CompilerParams(collective_id=0),
    )(A, B)


# device time: 240714 ns/iter; 3.3997x vs baseline; 3.3997x over previous
import jax
import jax.numpy as jnp
import numpy as np
from jax import lax
from jax.experimental import pallas as pl
from jax.experimental.pallas import tpu as pltpu

N_DEV = 32
M = 2048
N = 2048
CHUNK = M // N_DEV

BITS = (0, 3, 1, 4, 2)
RS_SIZE = (16, 8, 4, 2, 1)
RS_OFF = (0, 16, 24, 28, 30)


def _rowblock(v: int) -> int:
    vb = [(v >> k) & 1 for k in range(5)]
    return (vb[4] << 0) | (vb[2] << 1) | (vb[0] << 2) | (vb[3] << 3) | (vb[1] << 4)


def kernel(A, B):
    k = A.shape[1]
    a16 = A.astype(jnp.bfloat16)
    b16 = B.astype(jnp.bfloat16)
    perm = np.concatenate(
        [np.arange(CHUNK) + _rowblock(v) * CHUNK for v in range(N_DEV)]
    )
    a16v = a16[perm, :]

    def body(a_ref, b_ref, out_ref, p_ref, rs_ref, gat_ref, send_sems, recv_sems):
        my = lax.axis_index("i")
        u = 0
        for j, b in enumerate(BITS):
            u = u | (((my >> b) & 1) << (4 - j))
        partners = [my ^ (1 << b) for b in BITS]

        barrier = pltpu.get_barrier_semaphore()
        for pt in partners:
            pl.semaphore_signal(barrier, inc=1, device_id=(pt,),
                                device_id_type=pl.DeviceIdType.MESH)
        pl.semaphore_wait(barrier, 5)

        for i in range(4):
            rows = pl.ds(i * (M // 4), M // 4)
            p_ref[rows, :] = jnp.dot(
                a_ref[rows, :], b_ref[:, :], preferred_element_type=jnp.float32
            ).astype(jnp.bfloat16)

        base = 0
        for j in range(5):
            half = RS_SIZE[j]
            mybit = (u >> (4 - j)) & 1
            send_off = base + (1 - mybit) * half
            new_base = base + mybit * half
            stage_sl = pl.ds(RS_OFF[j] * CHUNK, half * CHUNK)
            rdma = pltpu.make_async_remote_copy(
                src_ref=p_ref.at[pl.ds(send_off * CHUNK, half * CHUNK), :],
                dst_ref=rs_ref.at[stage_sl, :],
                send_sem=send_sems.at[j],
                recv_sem=recv_sems.at[j],
                device_id=(partners[j],),
                device_id_type=pl.DeviceIdType.MESH,
            )
            rdma.start()
            rdma.wait_recv()
            keep_sl = pl.ds(new_base * CHUNK, half * CHUNK)
            p_ref[keep_sl, :] = p_ref[keep_sl, :] + rs_ref[stage_sl, :]
            rdma.wait_send()
            base = new_base

        mine_sl = pl.ds(base * CHUNK, CHUNK)
        gat_ref[mine_sl, :] = jnp.maximum(p_ref[mine_sl, :], 0)

        for j in range(5):
            size = 1 << j
            cb = (u >> j) << j
            blk_sl = pl.ds(cb * CHUNK, size * CHUNK)
            rdma = pltpu.make_async_remote_copy(
                src_ref=gat_ref.at[blk_sl, :],
                dst_ref=gat_ref.at[blk_sl, :],
                send_sem=send_sems.at[5 + j],
                recv_sem=recv_sems.at[5 + j],
                device_id=(partners[4 - j],),
                device_id_type=pl.DeviceIdType.MESH,
            )
            rdma.start()
            rdma.wait_recv()
            rdma.wait_send()

        for v in range(N_DEV):
            out_ref[pl.ds(_rowblock(v) * CHUNK, CHUNK), :] = (
                gat_ref[pl.ds(v * CHUNK, CHUNK), :]
            )

    return pl.pallas_call(
        body,
        out_shape=jax.ShapeDtypeStruct((M, N), jnp.bfloat16),
        in_specs=[
            pl.BlockSpec(memory_space=pltpu.VMEM),
            pl.BlockSpec(memory_space=pltpu.VMEM),
        ],
        out_specs=pl.BlockSpec(memory_space=pltpu.VMEM),
        scratch_shapes=[
            pltpu.VMEM((M, N), jnp.bfloat16),
            pltpu.VMEM((31 * CHUNK, N), jnp.bfloat16),
            pltpu.VMEM((M, N), jnp.bfloat16),
            pltpu.SemaphoreType.DMA((10,)),
            pltpu.SemaphoreType.DMA((10,)),
        ],
        compiler_params=pltpu.CompilerParams(collective_id=0),
    )(a16v, b16)


# device time: 170444 ns/iter; 4.8013x vs baseline; 1.4123x over previous
import jax
import jax.numpy as jnp
import numpy as np
from jax import lax
from jax.experimental import pallas as pl
from jax.experimental.pallas import tpu as pltpu

N_DEV = 32
M = 2048
N = 2048
CHUNK = M // N_DEV
HALF_N = N // 2

BITS_A = (0, 3, 1, 4, 2)
BITS_B = (3, 0, 4, 2, 1)
RS_SIZE = (16, 8, 4, 2, 1)
RS_OFF = (0, 16, 24, 28, 30)


def _rowblock(v: int, bits) -> int:
    vb = [(v >> k) & 1 for k in range(5)]
    p = 0
    for j, b in enumerate(bits):
        p |= vb[4 - j] << b
    return p


def _perm(bits) -> np.ndarray:
    return np.concatenate(
        [np.arange(CHUNK) + _rowblock(v, bits) * CHUNK for v in range(N_DEV)]
    )


def kernel(A, B):
    a16 = A.astype(jnp.bfloat16)
    b16 = B.astype(jnp.bfloat16)
    a16va = a16[_perm(BITS_A), :]
    a16vb = a16[_perm(BITS_B), :]

    def body(aa_ref, ab_ref, b_ref, out_ref,
             pa_ref, pb_ref, rsa_ref, rsb_ref, gata_ref, gatb_ref,
             send_sems, recv_sems):
        my = lax.axis_index("i")

        def virt(bits):
            u = 0
            for j, b in enumerate(bits):
                u = u | (((my >> b) & 1) << (4 - j))
            return u

        ua = virt(BITS_A)
        ub = virt(BITS_B)
        partners_a = [my ^ (1 << b) for b in BITS_A]
        partners_b = [my ^ (1 << b) for b in BITS_B]

        barrier = pltpu.get_barrier_semaphore()
        for pt in partners_a:
            pl.semaphore_signal(barrier, inc=1, device_id=(pt,),
                                device_id_type=pl.DeviceIdType.MESH)
        pl.semaphore_wait(barrier, 5)

        for i in range(4):
            rows = pl.ds(i * (M // 4), M // 4)
            pa_ref[rows, :] = jnp.dot(
                aa_ref[rows, :], b_ref[:, :HALF_N],
                preferred_element_type=jnp.float32,
            ).astype(jnp.bfloat16)
            pb_ref[rows, :] = jnp.dot(
                ab_ref[rows, :], b_ref[:, HALF_N:],
                preferred_element_type=jnp.float32,
            ).astype(jnp.bfloat16)

        def rs_rdma(j, p_ref, rs_ref, u, partner, sem_off):
            half = RS_SIZE[j]
            mybit = (u >> (4 - j)) & 1
            base = (u >> (5 - j)) << (5 - j) if j > 0 else 0
            send_off = base + (1 - mybit) * half
            new_base = base + mybit * half
            stage_sl = pl.ds(RS_OFF[j] * CHUNK, half * CHUNK)
            rdma = pltpu.make_async_remote_copy(
                src_ref=p_ref.at[pl.ds(send_off * CHUNK, half * CHUNK), :],
                dst_ref=rs_ref.at[stage_sl, :],
                send_sem=send_sems.at[sem_off + j],
                recv_sem=recv_sems.at[sem_off + j],
                device_id=(partner,),
                device_id_type=pl.DeviceIdType.MESH,
            )
            return rdma, new_base, stage_sl

        for j in range(5):
            rda, base_a, sl_a = rs_rdma(j, pa_ref, rsa_ref, ua, partners_a[j], 0)
            rda.start()
            rdb, base_b, sl_b = rs_rdma(j, pb_ref, rsb_ref, ub, partners_b[j], 10)
            rdb.start()
            half = RS_SIZE[j]
            rda.wait_recv()
            keep_a = pl.ds(base_a * CHUNK, half * CHUNK)
            pa_ref[keep_a, :] = pa_ref[keep_a, :] + rsa_ref[sl_a, :]
            rdb.wait_recv()
            keep_b = pl.ds(base_b * CHUNK, half * CHUNK)
            pb_ref[keep_b, :] = pb_ref[keep_b, :] + rsb_ref[sl_b, :]
            rda.wait_send()
            rdb.wait_send()

        mine_a = pl.ds(ua * CHUNK, CHUNK)
        gata_ref[mine_a, :] = jnp.maximum(pa_ref[mine_a, :], 0)
        mine_b = pl.ds(ub * CHUNK, CHUNK)
        gatb_ref[mine_b, :] = jnp.maximum(pb_ref[mine_b, :], 0)

        def ag_rdma(j, gat_ref, u, partner, sem_off):
            size = 1 << j
            cb = (u >> j) << j
            blk_sl = pl.ds(cb * CHUNK, size * CHUNK)
            return pltpu.make_async_remote_copy(
                src_ref=gat_ref.at[blk_sl, :],
                dst_ref=gat_ref.at[blk_sl, :],
                send_sem=send_sems.at[sem_off + 5 + j],
                recv_sem=recv_sems.at[sem_off + 5 + j],
                device_id=(partner,),
                device_id_type=pl.DeviceIdType.MESH,
            )

        for j in range(5):
            rda = ag_rdma(j, gata_ref, ua, partners_a[4 - j], 0)
            rda.start()
            rdb = ag_rdma(j, gatb_ref, ub, partners_b[4 - j], 10)
            rdb.start()
            rda.wait_recv()
            rdb.wait_recv()
            rda.wait_send()
            rdb.wait_send()

        for v in range(N_DEV):
            out_ref[pl.ds(_rowblock(v, BITS_A) * CHUNK, CHUNK), :HALF_N] = (
                gata_ref[pl.ds(v * CHUNK, CHUNK), :]
            )
            out_ref[pl.ds(_rowblock(v, BITS_B) * CHUNK, CHUNK), HALF_N:] = (
                gatb_ref[pl.ds(v * CHUNK, CHUNK), :]
            )

    return pl.pallas_call(
        body,
        out_shape=jax.ShapeDtypeStruct((M, N), jnp.bfloat16),
        in_specs=[
            pl.BlockSpec(memory_space=pltpu.VMEM),
            pl.BlockSpec(memory_space=pltpu.VMEM),
            pl.BlockSpec(memory_space=pltpu.VMEM),
        ],
        out_specs=pl.BlockSpec(memory_space=pltpu.VMEM),
        scratch_shapes=[
            pltpu.VMEM((M, HALF_N), jnp.bfloat16),
            pltpu.VMEM((M, HALF_N), jnp.bfloat16),
            pltpu.VMEM((31 * CHUNK, HALF_N), jnp.bfloat16),
            pltpu.VMEM((31 * CHUNK, HALF_N), jnp.bfloat16),
            pltpu.VMEM((M, HALF_N), jnp.bfloat16),
            pltpu.VMEM((M, HALF_N), jnp.bfloat16),
            pltpu.SemaphoreType.DMA((20,)),
            pltpu.SemaphoreType.DMA((20,)),
        ],
        compiler_params=pltpu.CompilerParams(collective_id=0),
    )(a16va, a16vb, b16)


# device time: 150481 ns/iter; 5.4382x vs baseline; 1.1327x over previous
import jax
import jax.numpy as jnp
import numpy as np
from jax import lax
from jax.experimental import pallas as pl
from jax.experimental.pallas import tpu as pltpu

N_DEV = 32
M = 2048
N = 2048
CHUNK = M // N_DEV
HALF_N = N // 2

BITS_A = (0, 3, 1, 4, 2)
BITS_B = (3, 0, 4, 2, 1)
RS_SIZE = (16, 8, 4, 2, 1)
RS_OFF = (0, 16, 24, 28, 30)


def _rowblock(v: int, bits) -> int:
    vb = [(v >> k) & 1 for k in range(5)]
    p = 0
    for j, b in enumerate(bits):
        p |= vb[4 - j] << b
    return p


def _perm(bits) -> np.ndarray:
    return np.concatenate(
        [np.arange(CHUNK) + _rowblock(v, bits) * CHUNK for v in range(N_DEV)]
    )


def kernel(A, B):
    a16 = A.astype(jnp.bfloat16)
    b16 = B.astype(jnp.bfloat16)

    def body(a_ref, b_ref, out_ref,
             pa_ref, pb_ref, rsa_ref, rsb_ref, gata_ref, gatb_ref,
             send_sems, recv_sems):
        my = lax.axis_index("i")

        def virt(bits):
            u = 0
            for j, b in enumerate(bits):
                u = u | (((my >> b) & 1) << (4 - j))
            return u

        ua = virt(BITS_A)
        ub = virt(BITS_B)
        partners_a = [my ^ (1 << b) for b in BITS_A]
        partners_b = [my ^ (1 << b) for b in BITS_B]

        barrier = pltpu.get_barrier_semaphore()
        for pt in partners_a:
            pl.semaphore_signal(barrier, inc=1, device_id=(pt,),
                                device_id_type=pl.DeviceIdType.MESH)
        pl.semaphore_wait(barrier, 5)

        def matmul_part(p_ref, bits, col_lo):
            for i in range(4):
                ablk = jnp.concatenate(
                    [
                        a_ref[pl.ds(_rowblock(v, bits) * CHUNK, CHUNK), :]
                        for v in range(8 * i, 8 * i + 8)
                    ],
                    axis=0,
                )
                p_ref[pl.ds(i * (M // 4), M // 4), :] = jnp.dot(
                    ablk, b_ref[:, col_lo:col_lo + HALF_N],
                    preferred_element_type=jnp.float32,
                ).astype(jnp.bfloat16)

        def rs_rdma(j, p_ref, rs_ref, u, partner, sem_off):
            half = RS_SIZE[j]
            mybit = (u >> (4 - j)) & 1
            base = (u >> (5 - j)) << (5 - j) if j > 0 else 0
            send_off = base + (1 - mybit) * half
            new_base = base + mybit * half
            stage_sl = pl.ds(RS_OFF[j] * CHUNK, half * CHUNK)
            rdma = pltpu.make_async_remote_copy(
                src_ref=p_ref.at[pl.ds(send_off * CHUNK, half * CHUNK), :],
                dst_ref=rs_ref.at[stage_sl, :],
                send_sem=send_sems.at[sem_off + j],
                recv_sem=recv_sems.at[sem_off + j],
                device_id=(partner,),
                device_id_type=pl.DeviceIdType.MESH,
            )
            return rdma, new_base, stage_sl

        matmul_part(pa_ref, BITS_A, 0)
        rda0 = rs_rdma(0, pa_ref, rsa_ref, ua, partners_a[0], 0)
        rda0[0].start()
        matmul_part(pb_ref, BITS_B, HALF_N)
        rdb0 = rs_rdma(0, pb_ref, rsb_ref, ub, partners_b[0], 10)
        rdb0[0].start()

        for j in range(5):
            if j == 0:
                rda, base_a, sl_a = rda0
                rdb, base_b, sl_b = rdb0
            else:
                rda, base_a, sl_a = rs_rdma(j, pa_ref, rsa_ref, ua,
                                            partners_a[j], 0)
                rda.start()
                rdb, base_b, sl_b = rs_rdma(j, pb_ref, rsb_ref, ub,
                                            partners_b[j], 10)
                rdb.start()
            half = RS_SIZE[j]
            rda.wait_recv()
            keep_a = pl.ds(base_a * CHUNK, half * CHUNK)
            pa_ref[keep_a, :] = pa_ref[keep_a, :] + rsa_ref[sl_a, :]
            rdb.wait_recv()
            keep_b = pl.ds(base_b * CHUNK, half * CHUNK)
            pb_ref[keep_b, :] = pb_ref[keep_b, :] + rsb_ref[sl_b, :]
            rda.wait_send()
            rdb.wait_send()

        mine_a = pl.ds(ua * CHUNK, CHUNK)
        gata_ref[mine_a, :] = jnp.maximum(pa_ref[mine_a, :], 0)
        mine_b = pl.ds(ub * CHUNK, CHUNK)
        gatb_ref[mine_b, :] = jnp.maximum(pb_ref[mine_b, :], 0)

        def ag_rdma(j, gat_ref, u, partner, sem_off):
            size = 1 << j
            cb = (u >> j) << j
            blk_sl = pl.ds(cb * CHUNK, size * CHUNK)
            return pltpu.make_async_remote_copy(
                src_ref=gat_ref.at[blk_sl, :],
                dst_ref=gat_ref.at[blk_sl, :],
                send_sem=send_sems.at[sem_off + 5 + j],
                recv_sem=recv_sems.at[sem_off + 5 + j],
                device_id=(partner,),
                device_id_type=pl.DeviceIdType.MESH,
            )

        for j in range(5):
            rda = ag_rdma(j, gata_ref, ua, partners_a[4 - j], 0)
            rda.start()
            rdb = ag_rdma(j, gatb_ref, ub, partners_b[4 - j], 10)
            rdb.start()
            rda.wait_recv()
            rdb.wait_recv()
            rda.wait_send()
            rdb.wait_send()

        for v in range(N_DEV):
            out_ref[pl.ds(_rowblock(v, BITS_A) * CHUNK, CHUNK), :HALF_N] = (
                gata_ref[pl.ds(v * CHUNK, CHUNK), :]
            )
            out_ref[pl.ds(_rowblock(v, BITS_B) * CHUNK, CHUNK), HALF_N:] = (
                gatb_ref[pl.ds(v * CHUNK, CHUNK), :]
            )

    return pl.pallas_call(
        body,
        out_shape=jax.ShapeDtypeStruct((M, N), jnp.bfloat16),
        in_specs=[
            pl.BlockSpec(memory_space=pltpu.VMEM),
            pl.BlockSpec(memory_space=pltpu.VMEM),
        ],
        out_specs=pl.BlockSpec(memory_space=pltpu.VMEM),
        scratch_shapes=[
            pltpu.VMEM((M, HALF_N), jnp.bfloat16),
            pltpu.VMEM((M, HALF_N), jnp.bfloat16),
            pltpu.VMEM((31 * CHUNK, HALF_N), jnp.bfloat16),
            pltpu.VMEM((31 * CHUNK, HALF_N), jnp.bfloat16),
            pltpu.VMEM((M, HALF_N), jnp.bfloat16),
            pltpu.VMEM((M, HALF_N), jnp.bfloat16),
            pltpu.SemaphoreType.DMA((20,)),
            pltpu.SemaphoreType.DMA((20,)),
        ],
        compiler_params=pltpu.CompilerParams(collective_id=0),
    )(a16, b16)


# device time: 143335 ns/iter; 5.7094x vs baseline; 1.0499x over previous
import jax
import jax.numpy as jnp
from jax import lax
from jax.experimental import pallas as pl
from jax.experimental.pallas import tpu as pltpu

N_DEV = 32
M = 2048
N = 2048
CHUNK = M // N_DEV
HALF_N = N // 2

BITS_A = (0, 3, 1, 4, 2)
BITS_B = (3, 0, 4, 2, 1)
RS_SIZE = (16, 8, 4, 2, 1)
RS_OFF = (0, 16, 24, 28, 30)


def _rowblock(v, bits):
    p = 0
    for j, b in enumerate(bits):
        p = p | (((v >> (4 - j)) & 1) << b)
    return p


def kernel(A, B):
    a16 = A.astype(jnp.bfloat16)
    b16 = B.astype(jnp.bfloat16)

    def body(a_ref, b_ref, out_ref,
             pa_ref, pb_ref, rsa_ref, rsb_ref, gata_ref, gatb_ref,
             send_sems, recv_sems):
        my = lax.axis_index("i")

        def virt(bits):
            u = 0
            for j, b in enumerate(bits):
                u = u | (((my >> b) & 1) << (4 - j))
            return u

        ua = virt(BITS_A)
        ub = virt(BITS_B)
        partners_a = [my ^ (1 << b) for b in BITS_A]
        partners_b = [my ^ (1 << b) for b in BITS_B]

        def rs_offsets(u):
            soff, nbase = [], []
            for j in range(5):
                half = RS_SIZE[j]
                mybit = (u >> (4 - j)) & 1
                base = (u >> (5 - j)) << (5 - j)
                soff.append(base + (1 - mybit) * half)
                nbase.append(base + mybit * half)
            return soff, nbase

        soff_a, nbase_a = rs_offsets(ua)
        soff_b, nbase_b = rs_offsets(ub)

        barrier = pltpu.get_barrier_semaphore()
        for pt in partners_a:
            pl.semaphore_signal(barrier, inc=1, device_id=(pt,),
                                device_id_type=pl.DeviceIdType.MESH)
        pl.semaphore_wait(barrier, 5)

        def matmul_half(p_ref, bits, col_lo, vbase):
            for i in range(2):
                v0 = vbase + 8 * i
                ablk = jnp.concatenate(
                    [
                        a_ref[pl.ds(_rowblock(v0 + t, bits) * CHUNK, CHUNK), :]
                        for t in range(8)
                    ],
                    axis=0,
                )
                p_ref[pl.ds(v0 * CHUNK, 8 * CHUNK), :] = jnp.dot(
                    ablk, b_ref[:, pl.ds(col_lo, HALF_N)],
                    preferred_element_type=jnp.float32,
                ).astype(jnp.bfloat16)

        def rs_rdma(j, p_ref, rs_ref, soff, partner, sem_off):
            half = RS_SIZE[j]
            return pltpu.make_async_remote_copy(
                src_ref=p_ref.at[pl.ds(soff[j] * CHUNK, half * CHUNK), :],
                dst_ref=rs_ref.at[pl.ds(RS_OFF[j] * CHUNK, half * CHUNK), :],
                send_sem=send_sems.at[sem_off + j],
                recv_sem=recv_sems.at[sem_off + j],
                device_id=(partner,),
                device_id_type=pl.DeviceIdType.MESH,
            )

        def acc_sub(p_ref, rs_ref, j, nb_j, x, length):
            dst = pl.ds(x * CHUNK, length * CHUNK)
            src = pl.ds((RS_OFF[j] + x - nb_j) * CHUNK, length * CHUNK)
            p_ref[dst, :] = p_ref[dst, :] + rs_ref[src, :]

        def ag_rdma(j, gat_ref, u, partner, sem_off):
            size = 1 << j
            cb = (u >> j) << j
            blk_sl = pl.ds(cb * CHUNK, size * CHUNK)
            return pltpu.make_async_remote_copy(
                src_ref=gat_ref.at[blk_sl, :],
                dst_ref=gat_ref.at[blk_sl, :],
                send_sem=send_sems.at[sem_off + 5 + j],
                recv_sem=recv_sems.at[sem_off + 5 + j],
                device_id=(partner,),
                device_id_type=pl.DeviceIdType.MESH,
            )

        def scatter(gat_ref, bits, col_lo, vbase, n):
            for t in range(n):
                v = vbase + t
                out_ref[pl.ds(_rowblock(v, bits) * CHUNK, CHUNK),
                        pl.ds(col_lo, HALF_N)] = (
                    gat_ref[pl.ds(v * CHUNK, CHUNK), :]
                )

        matmul_half(pa_ref, BITS_A, 0, soff_a[0])
        rda = rs_rdma(0, pa_ref, rsa_ref, soff_a, partners_a[0], 0)
        rda.start()
        matmul_half(pb_ref, BITS_B, HALF_N, soff_b[0])
        rdb = rs_rdma(0, pb_ref, rsb_ref, soff_b, partners_b[0], 10)
        rdb.start()
        matmul_half(pa_ref, BITS_A, 0, nbase_a[0])
        matmul_half(pb_ref, BITS_B, HALF_N, nbase_b[0])

        for j in range(4):
            nh = RS_SIZE[j + 1]
            rda.wait_recv()
            acc_sub(pa_ref, rsa_ref, j, nbase_a[j], soff_a[j + 1], nh)
            rda_n = rs_rdma(j + 1, pa_ref, rsa_ref, soff_a, partners_a[j + 1], 0)
            rda_n.start()
            rdb.wait_recv()
            acc_sub(pb_ref, rsb_ref, j, nbase_b[j], soff_b[j + 1], nh)
            rdb_n = rs_rdma(j + 1, pb_ref, rsb_ref, soff_b, partners_b[j + 1], 10)
            rdb_n.start()
            acc_sub(pa_ref, rsa_ref, j, nbase_a[j], nbase_a[j + 1], nh)
            acc_sub(pb_ref, rsb_ref, j, nbase_b[j], nbase_b[j + 1], nh)
            rda.wait_send()
            rdb.wait_send()
            rda, rdb = rda_n, rdb_n

        rda.wait_recv()
        acc_sub(pa_ref, rsa_ref, 4, nbase_a[4], nbase_a[4], 1)
        mine_a = pl.ds(ua * CHUNK, CHUNK)
        gata_ref[mine_a, :] = jnp.maximum(pa_ref[mine_a, :], 0)
        rga = ag_rdma(0, gata_ref, ua, partners_a[4], 0)
        rga.start()
        rdb.wait_recv()
        acc_sub(pb_ref, rsb_ref, 4, nbase_b[4], nbase_b[4], 1)
        mine_b = pl.ds(ub * CHUNK, CHUNK)
        gatb_ref[mine_b, :] = jnp.maximum(pb_ref[mine_b, :], 0)
        rgb = ag_rdma(0, gatb_ref, ub, partners_b[4], 10)
        rgb.start()
        rda.wait_send()
        rdb.wait_send()

        for j in range(5):
            size = 1 << j
            if j == 0:
                scatter(gata_ref, BITS_A, 0, ua, 1)
                scatter(gatb_ref, BITS_B, HALF_N, ub, 1)
            else:
                scatter(gata_ref, BITS_A, 0,
                        ((ua >> (j - 1)) << (j - 1)) ^ (1 << (j - 1)),
                        size // 2)
                scatter(gatb_ref, BITS_B, HALF_N,
                        ((ub >> (j - 1)) << (j - 1)) ^ (1 << (j - 1)),
                        size // 2)
            rga.wait_recv()
            if j < 4:
                rga_n = ag_rdma(j + 1, gata_ref, ua, partners_a[4 - j - 1], 0)
                rga_n.start()
            rgb.wait_recv()
            if j < 4:
                rgb_n = ag_rdma(j + 1, gatb_ref, ub, partners_b[4 - j - 1], 10)
                rgb_n.start()
            rga.wait_send()
            rgb.wait_send()
            if j < 4:
                rga, rgb = rga_n, rgb_n

        scatter(gata_ref, BITS_A, 0, ((ua >> 4) << 4) ^ 16, 16)
        scatter(gatb_ref, BITS_B, HALF_N, ((ub >> 4) << 4) ^ 16, 16)

    return pl.pallas_call(
        body,
        out_shape=jax.ShapeDtypeStruct((M, N), jnp.bfloat16),
        in_specs=[
            pl.BlockSpec(memory_space=pltpu.VMEM),
            pl.BlockSpec(memory_space=pltpu.VMEM),
        ],
        out_specs=pl.BlockSpec(memory_space=pltpu.VMEM),
        scratch_shapes=[
            pltpu.VMEM((M, HALF_N), jnp.bfloat16),
            pltpu.VMEM((M, HALF_N), jnp.bfloat16),
            pltpu.VMEM((31 * CHUNK, HALF_N), jnp.bfloat16),
            pltpu.VMEM((31 * CHUNK, HALF_N), jnp.bfloat16),
            pltpu.VMEM((M, HALF_N), jnp.bfloat16),
            pltpu.VMEM((M, HALF_N), jnp.bfloat16),
            pltpu.SemaphoreType.DMA((20,)),
            pltpu.SemaphoreType.DMA((20,)),
        ],
        compiler_params=pltpu.CompilerParams(collective_id=0),
    )(a16, b16)


# device time: 116562 ns/iter; 7.0207x vs baseline; 1.2297x over previous
import jax
import jax.numpy as jnp
from jax import lax
from jax.experimental import pallas as pl
from jax.experimental.pallas import tpu as pltpu

N_DEV = 32
M = 2048
N = 2048
CHUNK = M // N_DEV

MASKS_A = (1, 8, 2, 16, 4)
DUALS_A = MASKS_A
MASKS_B = (8, 1, 16, 4, 2)
DUALS_B = MASKS_B
MASKS_C = (3, 4, 2, 8, 16)
DUALS_C = (1, 4, 3, 8, 16)
COLS = (768, 768, 512)
COL_OFF = (0, 768, 1536)

RS_SIZE = (16, 8, 4, 2, 1)
RS_OFF = (0, 16, 24, 28, 30)


def _parity(x):
    t = x ^ (x >> 1)
    t = t ^ (t >> 2)
    t = t ^ (t >> 4)
    return t & 1


def _rowblock(v, masks):
    p = 0
    for j, m in enumerate(masks):
        p = p ^ (m * ((v >> (4 - j)) & 1))
    return p


def kernel(A, B):
    a16 = A.astype(jnp.bfloat16)
    b16 = B.astype(jnp.bfloat16)

    def body(a_ref, b_ref, out_ref,
             pa_ref, pb_ref, pc_ref, rsa_ref, rsb_ref, rsc_ref,
             gata_ref, gatb_ref, gatc_ref, send_sems, recv_sems):
        my = lax.axis_index("i")

        def virt(duals):
            u = 0
            for j, d in enumerate(duals):
                u = u | (_parity(my & d) << (4 - j))
            return u

        def rs_offsets(u):
            soff, nbase = [], []
            for j in range(5):
                half = RS_SIZE[j]
                mybit = (u >> (4 - j)) & 1
                base = (u >> (5 - j)) << (5 - j)
                soff.append(base + (1 - mybit) * half)
                nbase.append(base + mybit * half)
            return soff, nbase

        class Part:
            def __init__(self, masks, duals, p_ref, rs_ref, gat_ref,
                         col_off, cols, sem_off):
                self.masks = masks
                self.u = virt(duals)
                self.partners = [my ^ m for m in masks]
                self.p_ref = p_ref
                self.rs_ref = rs_ref
                self.gat_ref = gat_ref
                self.col_off = col_off
                self.cols = cols
                self.sem_off = sem_off
                self.soff, self.nbase = rs_offsets(self.u)
                self.rd = None

        parts = [
            Part(MASKS_A, DUALS_A, pa_ref, rsa_ref, gata_ref,
                 COL_OFF[0], COLS[0], 0),
            Part(MASKS_B, DUALS_B, pb_ref, rsb_ref, gatb_ref,
                 COL_OFF[1], COLS[1], 10),
            Part(MASKS_C, DUALS_C, pc_ref, rsc_ref, gatc_ref,
                 COL_OFF[2], COLS[2], 20),
        ]

        barrier = pltpu.get_barrier_semaphore()
        for mk in (1, 2, 3, 4, 8, 16):
            pl.semaphore_signal(barrier, inc=1, device_id=(my ^ mk,),
                                device_id_type=pl.DeviceIdType.MESH)
        pl.semaphore_wait(barrier, 6)

        def matmul_half(pt, vbase):
            for i in range(2):
                v0 = vbase + 8 * i
                ablk = jnp.concatenate(
                    [
                        a_ref[pl.ds(_rowblock(v0 + t, pt.masks) * CHUNK,
                                    CHUNK), :]
                        for t in range(8)
                    ],
                    axis=0,
                )
                pt.p_ref[pl.ds(v0 * CHUNK, 8 * CHUNK), :] = jnp.dot(
                    ablk, b_ref[:, pl.ds(pt.col_off, pt.cols)],
                    preferred_element_type=jnp.float32,
                ).astype(jnp.bfloat16)

        def rs_rdma(pt, j):
            half = RS_SIZE[j]
            return pltpu.make_async_remote_copy(
                src_ref=pt.p_ref.at[pl.ds(pt.soff[j] * CHUNK,
                                          half * CHUNK), :],
                dst_ref=pt.rs_ref.at[pl.ds(RS_OFF[j] * CHUNK,
                                           half * CHUNK), :],
                send_sem=send_sems.at[pt.sem_off + j],
                recv_sem=recv_sems.at[pt.sem_off + j],
                device_id=(pt.partners[j],),
                device_id_type=pl.DeviceIdType.MESH,
            )

        def acc_sub(pt, j, x, length):
            dst = pl.ds(x * CHUNK, length * CHUNK)
            src = pl.ds((RS_OFF[j] + x - pt.nbase[j]) * CHUNK,
                        length * CHUNK)
            pt.p_ref[dst, :] = pt.p_ref[dst, :] + pt.rs_ref[src, :]

        def ag_rdma(pt, j):
            size = 1 << j
            cb = (pt.u >> j) << j
            blk_sl = pl.ds(cb * CHUNK, size * CHUNK)
            return pltpu.make_async_remote_copy(
                src_ref=pt.gat_ref.at[blk_sl, :],
                dst_ref=pt.gat_ref.at[blk_sl, :],
                send_sem=send_sems.at[pt.sem_off + 5 + j],
                recv_sem=recv_sems.at[pt.sem_off + 5 + j],
                device_id=(pt.partners[4 - j],),
                device_id_type=pl.DeviceIdType.MESH,
            )

        def scatter(pt, vbase, n):
            for t in range(n):
                v = vbase + t
                out_ref[pl.ds(_rowblock(v, pt.masks) * CHUNK, CHUNK),
                        pl.ds(pt.col_off, pt.cols)] = (
                    pt.gat_ref[pl.ds(v * CHUNK, CHUNK), :]
                )

        for pt in parts:
            matmul_half(pt, pt.soff[0])
            pt.rd = rs_rdma(pt, 0)
            pt.rd.start()
        for pt in parts:
            matmul_half(pt, pt.nbase[0])

        for j in range(4):
            nh = RS_SIZE[j + 1]
            for pt in parts:
                pt.rd.wait_recv()
                acc_sub(pt, j, pt.soff[j + 1], nh)
                rd_n = rs_rdma(pt, j + 1)
                rd_n.start()
                pt.rd_prev, pt.rd = pt.rd, rd_n
            for pt in parts:
                acc_sub(pt, j, pt.nbase[j + 1], nh)
            for pt in parts:
                pt.rd_prev.wait_send()

        for pt in parts:
            pt.rd.wait_recv()
            acc_sub(pt, 4, pt.nbase[4], 1)
            mine = pl.ds(pt.u * CHUNK, CHUNK)
            pt.gat_ref[mine, :] = jnp.maximum(pt.p_ref[mine, :], 0)
            rd_n = ag_rdma(pt, 0)
            rd_n.start()
            pt.rd_prev, pt.rd = pt.rd, rd_n
        for pt in parts:
            pt.rd_prev.wait_send()

        for j in range(5):
            for pt in parts:
                if j == 0:
                    scatter(pt, pt.u, 1)
                else:
                    half = 1 << (j - 1)
                    scatter(pt, ((pt.u >> (j - 1)) << (j - 1)) ^ half, half)
            for pt in parts:
                pt.rd.wait_recv()
                if j < 4:
                    rd_n = ag_rdma(pt, j + 1)
                    rd_n.start()
                    pt.rd_prev, pt.rd = pt.rd, rd_n
                else:
                    pt.rd_prev = pt.rd
            for pt in parts:
                pt.rd_prev.wait_send()

        for pt in parts:
            scatter(pt, ((pt.u >> 4) << 4) ^ 16, 16)

    return pl.pallas_call(
        body,
        out_shape=jax.ShapeDtypeStruct((M, N), jnp.bfloat16),
        in_specs=[
            pl.BlockSpec(memory_space=pltpu.VMEM),
            pl.BlockSpec(memory_space=pltpu.VMEM),
        ],
        out_specs=pl.BlockSpec(memory_space=pltpu.VMEM),
        scratch_shapes=[
            pltpu.VMEM((M, COLS[0]), jnp.bfloat16),
            pltpu.VMEM((M, COLS[1]), jnp.bfloat16),
            pltpu.VMEM((M, COLS[2]), jnp.bfloat16),
            pltpu.VMEM((31 * CHUNK, COLS[0]), jnp.bfloat16),
            pltpu.VMEM((31 * CHUNK, COLS[1]), jnp.bfloat16),
            pltpu.VMEM((31 * CHUNK, COLS[2]), jnp.bfloat16),
            pltpu.VMEM((M, COLS[0]), jnp.bfloat16),
            pltpu.VMEM((M, COLS[1]), jnp.bfloat16),
            pltpu.VMEM((M, COLS[2]), jnp.bfloat16),
            pltpu.SemaphoreType.DMA((30,)),
            pltpu.SemaphoreType.DMA((30,)),
        ],
        compiler_params=pltpu.CompilerParams(collective_id=0),
    )(a16, b16)
